# baseline (speedup 1.0000x reference)
import functools

import jax
import jax.numpy as jnp
from jax import lax
from jax.experimental import pallas as pl
from jax.experimental.pallas import tpu as pltpu
from jax.experimental.pallas import tpu_sc as plsc

DOC_LEN = 16384
DIMS = 128
CLASSES = 1000

NC = 2
NS = 16
NW = NC * NS
PER_W = DOC_LEN // NW
LANES = 16
NVREG = DIMS // LANES
SEGS = ((0, 128), (128, 128), (256, 128), (384, 112), (496, 16))


def _sc_body(nums_hbm, emb_hbm, out_hbm, idx_v, rows_v, acc_v,
             sem0, sem1, sem2, sem3, sem4):
    wid = lax.axis_index("s") * NC + lax.axis_index("c")
    base = wid * PER_W
    sems = (sem0, sem1, sem2, sem3, sem4)
    first = SEGS[0][1]
    pltpu.sync_copy(nums_hbm.at[pl.ds(base, first)],
                    idx_v.at[pl.ds(0, first)])
    cps = [pltpu.async_copy(
        emb_hbm.at[idx_v.at[pl.ds(0, first)]],
        rows_v.at[pl.ds(0, first)], sems[0])]
    pltpu.sync_copy(nums_hbm.at[pl.ds(base + first, PER_W - first)],
                    idx_v.at[pl.ds(first, PER_W - first)])
    cps += [
        pltpu.async_copy(
            emb_hbm.at[idx_v.at[pl.ds(off, n)]],
            rows_v.at[pl.ds(off, n)], sems[i])
        for i, (off, n) in enumerate(SEGS) if i > 0
    ]
    accs = tuple(jnp.zeros((LANES,), jnp.float32) for _ in range(NVREG))
    for i, (off, n) in enumerate(SEGS):
        cps[i].wait()

        def row_body(r, a):
            return tuple(
                a[j] + rows_v[r, pl.ds(j * LANES, LANES)]
                for j in range(NVREG))

        accs = lax.fori_loop(off, off + n, row_body, accs)
    for j in range(NVREG):
        acc_v[pl.ds(j * LANES, LANES)] = accs[j]
    pltpu.sync_copy(acc_v, out_hbm.at[wid])


_sc_gather_sum = functools.partial(
    pl.kernel,
    mesh=plsc.VectorSubcoreMesh(core_axis_name="c", subcore_axis_name="s"),
    out_type=jax.ShapeDtypeStruct((NW, DIMS), jnp.float32),
    scratch_types=[
        pltpu.VMEM((PER_W,), jnp.int32),
        pltpu.VMEM((PER_W, DIMS), jnp.float32),
        pltpu.VMEM((DIMS,), jnp.float32),
        pltpu.SemaphoreType.DMA,
        pltpu.SemaphoreType.DMA,
        pltpu.SemaphoreType.DMA,
        pltpu.SemaphoreType.DMA,
        pltpu.SemaphoreType.DMA,
    ],
)(_sc_body)


def _tc_body(k_ref, part_ref, w_ref, b_ref, out_ref):
    doc = jnp.sum(part_ref[...], axis=0, keepdims=True)
    logits = jnp.dot(doc, w_ref[...],
                     preferred_element_type=jnp.float32) + b_ref[...]
    m = jnp.max(logits)
    lse = jnp.log(jnp.sum(jnp.exp(logits - m))) + m
    col = lax.broadcasted_iota(jnp.int32, (1, CLASSES), 1)
    sel = jnp.sum(jnp.where(col == k_ref[0], logits, 0.0))
    out_ref[0, 0] = lse - sel


def _tc_tail(karr, partials, w, b2):
    return pl.pallas_call(
        _tc_body,
        out_shape=jax.ShapeDtypeStruct((1, 1), jnp.float32),
        in_specs=[
            pl.BlockSpec(memory_space=pltpu.SMEM),
            pl.BlockSpec(memory_space=pltpu.VMEM),
            pl.BlockSpec(memory_space=pltpu.VMEM),
            pl.BlockSpec(memory_space=pltpu.VMEM),
        ],
        out_specs=pl.BlockSpec(memory_space=pltpu.SMEM),
    )(karr, partials, w, b2)


def kernel(nums, emb, W, b, k):
    partials = _sc_gather_sum(nums, emb)
    karr = jnp.asarray(k, jnp.int32).reshape(1)
    loss = _tc_tail(karr, partials, W, b.reshape(1, CLASSES))
    return loss[0, 0]


# speedup vs baseline: 1.0100x; 1.0100x over previous
import functools

import jax
import jax.numpy as jnp
from jax import lax
from jax.experimental import pallas as pl
from jax.experimental.pallas import tpu as pltpu
from jax.experimental.pallas import tpu_sc as plsc

DOC_LEN = 16384
DIMS = 128
CLASSES = 1000

NC = 2
NS = 16
NW = NC * NS
PER_W = DOC_LEN // NW
LANES = 16
NVREG = DIMS // LANES
SEGS = ((0, 128), (128, 128), (256, 128), (384, 112), (496, 16))


def _sc_body(nums_hbm, emb_hbm, out_hbm, idx_v, rows_v, acc_v,
             sem0, sem1, sem2, sem3, sem4):
    wid = lax.axis_index("s") * NC + lax.axis_index("c")
    base = wid * PER_W
    sems = (sem0, sem1, sem2, sem3, sem4)
    first = SEGS[0][1]
    pltpu.sync_copy(nums_hbm.at[pl.ds(base, first)],
                    idx_v.at[pl.ds(0, first)])
    cps = [pltpu.async_copy(
        emb_hbm.at[idx_v.at[pl.ds(0, first)]],
        rows_v.at[pl.ds(0, first)], sems[0])]
    pltpu.sync_copy(nums_hbm.at[pl.ds(base + first, PER_W - first)],
                    idx_v.at[pl.ds(first, PER_W - first)])
    cps += [
        pltpu.async_copy(
            emb_hbm.at[idx_v.at[pl.ds(off, n)]],
            rows_v.at[pl.ds(off, n)], sems[i])
        for i, (off, n) in enumerate(SEGS) if i > 0
    ]
    accs = tuple(jnp.zeros((LANES,), jnp.float32) for _ in range(NVREG))
    for i, (off, n) in enumerate(SEGS):
        cps[i].wait()

        def row_body(r, a):
            return tuple(
                a[j] + rows_v[r, pl.ds(j * LANES, LANES)]
                for j in range(NVREG))

        accs = lax.fori_loop(off, off + n, row_body, accs)
    for j in range(NVREG):
        acc_v[pl.ds(j * LANES, LANES)] = accs[j]
    pltpu.sync_copy(acc_v, out_hbm.at[wid])


_sc_gather_sum = functools.partial(
    pl.kernel,
    mesh=plsc.VectorSubcoreMesh(core_axis_name="c", subcore_axis_name="s"),
    out_type=jax.ShapeDtypeStruct((NW, DIMS), jnp.float32),
    scratch_types=[
        pltpu.VMEM((PER_W,), jnp.int32),
        pltpu.VMEM((PER_W, DIMS), jnp.float32),
        pltpu.VMEM((DIMS,), jnp.float32),
        pltpu.SemaphoreType.DMA,
        pltpu.SemaphoreType.DMA,
        pltpu.SemaphoreType.DMA,
        pltpu.SemaphoreType.DMA,
        pltpu.SemaphoreType.DMA,
    ],
)(_sc_body)


PAD_C = 1024


def _tc_body(k_ref, part_ref, w_ref, b_ref, out_ref):
    doc = jnp.sum(part_ref[...], axis=0, keepdims=True)
    logits = jnp.dot(doc, w_ref[...],
                     preferred_element_type=jnp.float32) + b_ref[...]
    m = jnp.max(logits)
    lse = jnp.log(jnp.sum(jnp.exp(logits - m))) + m
    col = lax.broadcasted_iota(jnp.int32, (1, PAD_C), 1)
    sel = jnp.sum(jnp.where(col == k_ref[0], logits, 0.0))
    out_ref[0, 0] = lse - sel


def _tc_tail(karr, partials, w, b2):
    return pl.pallas_call(
        _tc_body,
        out_shape=jax.ShapeDtypeStruct((1, 1), jnp.float32),
        in_specs=[
            pl.BlockSpec(memory_space=pltpu.SMEM),
            pl.BlockSpec(memory_space=pltpu.VMEM),
            pl.BlockSpec(memory_space=pltpu.VMEM),
            pl.BlockSpec(memory_space=pltpu.VMEM),
        ],
        out_specs=pl.BlockSpec(memory_space=pltpu.SMEM),
    )(karr, partials, w, b2)


def kernel(nums, emb, W, b, k):
    partials = _sc_gather_sum(nums, emb)
    c = W.shape[1]
    w_pad = jnp.pad(W, ((0, 0), (0, PAD_C - c)))
    b_pad = jnp.pad(b, (0, PAD_C - c),
                    constant_values=-1e30).reshape(1, PAD_C)
    karr = jnp.asarray(k, jnp.int32).reshape(1)
    loss = _tc_tail(karr, partials, w_pad, b_pad)
    return loss[0, 0]
